# baseline (speedup 1.0000x reference)
import functools

import jax
import jax.numpy as jnp
from jax import lax
from jax.experimental import pallas as pl
from jax.experimental.pallas import tpu as pltpu
from jax.experimental.pallas import tpu_sc as plsc

_N = 1_000_000
_DIM = 64
_NC = 2
_NS = 16
_NW = _NC * _NS
_CHUNK = 128
_MAIN_ITERS = 244
_MAIN_PER_W = _MAIN_ITERS * _CHUNK
_MAIN = _NW * _MAIN_PER_W
_TAIL_CHUNK = 64
_TAIL_WORKERS = (_N - _MAIN) // _TAIL_CHUNK
_NBUF = 6
_LEAD = 3
_ILEAD = 5
_OUTER = 40


@jax.jit
def _sc_gather(species, species2d, table):
    mesh = plsc.VectorSubcoreMesh(core_axis_name="c", subcore_axis_name="s")

    @functools.partial(
        pl.kernel,
        out_type=jax.ShapeDtypeStruct((_N, _DIM), jnp.float32),
        mesh=mesh,
        scratch_types=[
            [pltpu.VMEM((_CHUNK, _DIM), jnp.float32) for _ in range(_NBUF)],
            [pltpu.VMEM((_CHUNK,), jnp.int32) for _ in range(_NBUF)],
            pltpu.VMEM((_TAIL_CHUNK,), jnp.int32),
            [pltpu.SemaphoreType.DMA for _ in range(_NBUF)],
            [pltpu.SemaphoreType.DMA for _ in range(_NBUF)],
            [pltpu.SemaphoreType.DMA for _ in range(_NBUF)],
            pltpu.SemaphoreType.DMA,
        ],
        compiler_params=pltpu.CompilerParams(use_tc_tiling_on_sc=False),
    )
    def k(species_hbm, species2d_hbm, table_hbm, out_hbm, rows,
          idx_c, idx_t, isem, gsem, osem, sem):
        wid = lax.axis_index("s") * _NC + lax.axis_index("c")
        base_w = wid * _MAIN_PER_W

        def _idx_load(i, b):
            pltpu.make_async_copy(
                species_hbm.at[pl.ds(base_w + i * _CHUNK, _CHUNK)],
                idx_c[b], isem[b]).start()

        def _idx_wait(b):
            pltpu.make_async_copy(
                species_hbm.at[pl.ds(base_w, _CHUNK)], idx_c[b],
                isem[b]).wait()

        def _gather(i, b):
            _idx_wait(b)
            pltpu.make_async_copy(
                table_hbm.at[idx_c[b]], rows[b], gsem[b]).start()

        def _gather_wait(b):
            pltpu.make_async_copy(
                table_hbm.at[idx_c[b]], rows[b], gsem[b]).wait()

        def _write(i, b):
            pltpu.make_async_copy(
                rows[b], out_hbm.at[pl.ds(base_w + i * _CHUNK, _CHUNK)],
                osem[b]).start()

        def _write_wait(b):
            pltpu.make_async_copy(
                rows[b], out_hbm.at[pl.ds(base_w, _CHUNK)],
                osem[b]).wait()

        for b in range(_ILEAD):
            _idx_load(b, b)
        for b in range(_LEAD):
            _gather(b, b)

        def body(g, carry):
            for b in range(_NBUF):
                i = g * _NBUF + b
                _gather_wait(b)
                _write(i, b)
                ib = (b + _ILEAD) % _NBUF
                if b + _ILEAD < _NBUF * _OUTER + 4 - (_NBUF * (_OUTER - 1)):
                    _idx_load(i + _ILEAD, ib)
                else:
                    @pl.when(g < _OUTER - 1)
                    def _i():
                        _idx_load(i + _ILEAD, ib)
                jb = (b + _LEAD) % _NBUF
                if b < _LEAD:
                    @pl.when(g >= 1)
                    def _w():
                        _write_wait(jb)
                else:
                    _write_wait(jb)
                _gather(i + _LEAD, jb)
            return carry

        lax.fori_loop(0, _OUTER, body, 0)

        for i in range(_OUTER * _NBUF, _MAIN_ITERS):
            b = i % _NBUF
            if i >= _OUTER * _NBUF + _LEAD:
                _write_wait(b)
                _gather(i, b)
            _gather_wait(b)
            _write(i, b)

        peeled = [i % _NBUF for i in range(_OUTER * _NBUF + _LEAD, _MAIN_ITERS)]
        for i in range(_OUTER * _NBUF - _LEAD, _MAIN_ITERS):
            b = i % _NBUF
            if b not in peeled or i >= _OUTER * _NBUF:
                _write_wait(b)

        @pl.when(wid < _TAIL_WORKERS)
        def _tail():
            tb = _MAIN + wid * _TAIL_CHUNK
            rows_t = rows[0].at[pl.ds(0, _TAIL_CHUNK)]
            pltpu.sync_copy(species_hbm.at[pl.ds(tb, _TAIL_CHUNK)], idx_t)
            pltpu.async_copy(table_hbm.at[idx_t], rows_t, sem).wait()
            pltpu.sync_copy(rows_t, out_hbm.at[pl.ds(tb, _TAIL_CHUNK)])

    return k(species, species2d, table)


def kernel(species, conv_tensor):
    species2d = species[:_MAIN].reshape(_NW, _MAIN_ITERS, _CHUNK)
    return _sc_gather(species, species2d, conv_tensor.astype(jnp.float32))


# speedup vs baseline: 2.6825x; 2.6825x over previous
import functools

import jax
import jax.numpy as jnp
from jax import lax
from jax.experimental import pallas as pl
from jax.experimental.pallas import tpu as pltpu
from jax.experimental.pallas import tpu_sc as plsc

_N = 1_000_000
_DIM = 64
_NC = 2
_NS = 16
_NW = _NC * _NS
_CHUNK = 128
_MAIN_ITERS = 244
_MAIN_PER_W = _MAIN_ITERS * _CHUNK
_MAIN = _NW * _MAIN_PER_W
_TAIL_CHUNK = 64
_TAIL_WORKERS = (_N - _MAIN) // _TAIL_CHUNK
_NBUF = 4
_OUTER = _MAIN_ITERS // _NBUF


@jax.jit
def _sc_gather(species, species2d, table):
    mesh = plsc.VectorSubcoreMesh(core_axis_name="c", subcore_axis_name="s")

    @functools.partial(
        pl.kernel,
        out_type=jax.ShapeDtypeStruct((_N, _DIM), jnp.float32),
        mesh=mesh,
        scratch_types=[
            pltpu.VMEM_SHARED((119, _DIM), jnp.float32),
            pltpu.VMEM((_MAIN_ITERS, _CHUNK), jnp.int32),
            [pltpu.VMEM((_CHUNK, _DIM), jnp.float32) for _ in range(_NBUF)],
            [pltpu.VMEM((_CHUNK,), jnp.int32) for _ in range(_NBUF)],
            pltpu.VMEM((_TAIL_CHUNK,), jnp.int32),
            pltpu.VMEM((_TAIL_CHUNK, _DIM), jnp.float32),
            [pltpu.SemaphoreType.DMA for _ in range(_NBUF)],
            [pltpu.SemaphoreType.DMA for _ in range(_NBUF)],
            pltpu.SemaphoreType.DMA,
        ],
    )
    def k(species_hbm, species2d_hbm, table_hbm, out_hbm, table_sp, idx_v, rows,
          idx_c, idx_t, rows_t, gsem, osem, sem):
        wid = lax.axis_index("s") * _NC + lax.axis_index("c")
        base_w = wid * _MAIN_PER_W

        @pl.when(lax.axis_index("s") == 0)
        def _fill():
            pltpu.sync_copy(table_hbm, table_sp)

        plsc.subcore_barrier()

        pltpu.sync_copy(species2d_hbm.at[wid], idx_v)

        def _gather(i, b):
            for j in range(_CHUNK // 16):
                idx_c[b][pl.ds(j * 16, 16)] = idx_v[i, pl.ds(j * 16, 16)]
            pltpu.make_async_copy(
                table_sp.at[idx_c[b]], rows[b], gsem[b]).start()

        def _gather_wait(b):
            pltpu.make_async_copy(
                table_sp.at[idx_c[b]], rows[b], gsem[b]).wait()

        def _write(i, b):
            pltpu.make_async_copy(
                rows[b], out_hbm.at[pl.ds(base_w + i * _CHUNK, _CHUNK)],
                osem[b]).start()

        def _write_wait(b):
            pltpu.make_async_copy(
                rows[b], out_hbm.at[pl.ds(base_w, _CHUNK)],
                osem[b]).wait()

        _gather(0, 0)
        _gather(1, 1)

        def body(g, carry):
            for b in range(_NBUF):
                i = g * _NBUF + b
                _gather_wait(b)
                _write(i, b)
                jb = (b + 2) % _NBUF
                if b < 2:
                    @pl.when(g >= 1)
                    def _w():
                        _write_wait(jb)
                    _gather(i + 2, jb)
                else:
                    @pl.when(g < _OUTER - 1)
                    def _g():
                        _write_wait(jb)
                        _gather(i + 2, jb)
            return carry

        lax.fori_loop(0, _OUTER, body, 0)

        @pl.when(wid < _TAIL_WORKERS)
        def _tail():
            tb = _MAIN + wid * _TAIL_CHUNK
            pltpu.sync_copy(species_hbm.at[pl.ds(tb, _TAIL_CHUNK)], idx_t)
            pltpu.async_copy(table_sp.at[idx_t], rows_t, sem).wait()
            pltpu.sync_copy(rows_t, out_hbm.at[pl.ds(tb, _TAIL_CHUNK)])

        for b in range(_NBUF):
            _write_wait(b)

    return k(species, species2d, table)


def kernel(species, conv_tensor):
    species2d = species[:_MAIN].reshape(_NW, _MAIN_ITERS, _CHUNK)
    return _sc_gather(species, species2d, conv_tensor.astype(jnp.float32))
